# baseline (speedup 1.0000x reference)
import functools

import jax
import jax.numpy as jnp
from jax import lax
from jax.experimental import pallas as pl
from jax.experimental.pallas import tpu as pltpu
from jax.experimental.pallas import tpu_sc as plsc

_L = 16
_NC = 2
_NS = 16


def _segment_sum_sc(edges, dist_row, n_pad):
    nw = _NC * _NS
    n_e = dist_row.shape[1]
    ch = n_e // nw // 128 * 128
    tail = n_e - nw * ch
    mesh = plsc.VectorSubcoreMesh(core_axis_name="c", subcore_axis_name="s")

    @functools.partial(
        pl.kernel,
        mesh=mesh,
        compiler_params=pltpu.CompilerParams(needs_layout_passes=False),
        out_type=jax.ShapeDtypeStruct((nw, n_pad), jnp.float32),
        scratch_types=[
            pltpu.VMEM((2, ch), jnp.int32),
            pltpu.VMEM((ch,), jnp.float32),
            pltpu.VMEM((2, tail), jnp.int32),
            pltpu.VMEM((tail,), jnp.float32),
            pltpu.VMEM((n_pad,), jnp.float32),
            pltpu.SemaphoreType.DMA,
            pltpu.SemaphoreType.DMA,
            pltpu.SemaphoreType.DMA,
            pltpu.SemaphoreType.DMA,
        ],
    )
    def seg_sum(edges_hbm, dist_hbm, out_hbm,
                idx_v, val_v, idx_x, val_x, acc_v, sem1, sem2, sem3, sem4):
        wid = lax.axis_index("s") * _NC + lax.axis_index("c")
        base = wid * ch
        half = ch // 2
        cp_i0 = pltpu.async_copy(
            edges_hbm.at[:, pl.ds(base, half)], idx_v.at[:, pl.ds(0, half)],
            sem1)
        cp_v0 = pltpu.async_copy(
            dist_hbm.at[0, pl.ds(base, half)], val_v.at[pl.ds(0, half)], sem2)
        cp_i1 = pltpu.async_copy(
            edges_hbm.at[:, pl.ds(base + half, half)],
            idx_v.at[:, pl.ds(half, half)], sem3)
        cp_v1 = pltpu.async_copy(
            dist_hbm.at[0, pl.ds(base + half, half)],
            val_v.at[pl.ds(half, half)], sem4)

        @plsc.parallel_loop(0, n_pad, step=_L, unroll=8)
        def zero(i):
            acc_v[pl.ds(i, _L)] = jnp.zeros((_L,), jnp.float32)

        cp_i0.wait()
        cp_v0.wait()

        @plsc.parallel_loop(0, half, step=_L, unroll=8)
        def body0(i):
            plsc.addupdate_scatter(
                acc_v, [idx_v[0, pl.ds(i, _L)]], val_v[pl.ds(i, _L)]
            )

        cp_i1.wait()
        cp_v1.wait()

        @plsc.parallel_loop(half, ch, step=_L, unroll=8)
        def body1(i):
            plsc.addupdate_scatter(
                acc_v, [idx_v[0, pl.ds(i, _L)]], val_v[pl.ds(i, _L)]
            )

        @pl.when(wid == 0)
        def _():
            cpi = pltpu.async_copy(
                edges_hbm.at[:, pl.ds(nw * ch, tail)], idx_x, sem1)
            cpv = pltpu.async_copy(
                dist_hbm.at[0, pl.ds(nw * ch, tail)], val_x, sem2)
            cpi.wait()
            cpv.wait()

            @plsc.parallel_loop(0, tail, step=_L, unroll=8)
            def tail_body(i):
                plsc.addupdate_scatter(
                    acc_v, [idx_x[0, pl.ds(i, _L)]], val_x[pl.ds(i, _L)]
                )

        pltpu.sync_copy(acc_v, out_hbm.at[wid])

    return seg_sum(edges, dist_row)


def _node_update_tc(h, parts, Wn1, b1, Wn2, b2):
    n, d = h.shape
    nw = parts.shape[0]
    B = 1024
    grid = (pl.cdiv(n, B),)

    def body(h_ref, p_ref, Wn1_ref, b1_ref, Wn2_ref, b2_ref, out_ref):
        hb = h_ref[...]
        w1b = Wn1_ref[d:d + 1, :] * 0.01
        C = jnp.broadcast_to(w1b, (nw, d))
        t = jnp.dot(hb, Wn1_ref[:d, :], preferred_element_type=jnp.float32)
        t = t + lax.dot_general(
            p_ref[...], C, (((0,), (0,)), ((), ())),
            preferred_element_type=jnp.float32,
        )
        t = t + b1_ref[...]
        t = t * jax.nn.sigmoid(t)
        o = jnp.dot(t, Wn2_ref[...], preferred_element_type=jnp.float32)
        out_ref[...] = o + b2_ref[...] + hb

    return pl.pallas_call(
        body,
        grid=grid,
        in_specs=[
            pl.BlockSpec((B, d), lambda i: (i, 0)),
            pl.BlockSpec((nw, B), lambda i: (0, i)),
            pl.BlockSpec((d + 1, d), lambda i: (0, 0)),
            pl.BlockSpec((1, d), lambda i: (0, 0)),
            pl.BlockSpec((d, d), lambda i: (0, 0)),
            pl.BlockSpec((1, d), lambda i: (0, 0)),
        ],
        out_specs=pl.BlockSpec((B, d), lambda i: (i, 0)),
        out_shape=jax.ShapeDtypeStruct((n, d), jnp.float32),
    )(h, parts, Wn1, b1.reshape(1, d), Wn2, b2.reshape(1, d))


def kernel(h, edges, distances, W_edg1, b_edg1, W_edg2, b_edg2,
           W_edgi, b_edgi, W_node1, b_node1, W_node2, b_node2):
    n_nodes, d = h.shape
    n_pad = ((n_nodes + 2047) // 2048) * 2048
    parts = _segment_sum_sc(edges.astype(jnp.int32),
                            distances.reshape(1, -1), n_pad)
    return _node_update_tc(h, parts, W_node1, b_node1, W_node2, b_node2)


# speedup vs baseline: 1.0747x; 1.0747x over previous
import functools

import jax
import jax.numpy as jnp
from jax import lax
from jax.experimental import pallas as pl
from jax.experimental.pallas import tpu as pltpu
from jax.experimental.pallas import tpu_sc as plsc

_L = 16
_NC = 2
_NS = 16


def _segment_sum_sc(edges, dist_row, n_pad):
    nw = _NC * _NS
    n_e = dist_row.shape[1]
    ch = n_e // nw // 128 * 128
    tail = n_e - nw * ch
    mesh = plsc.VectorSubcoreMesh(core_axis_name="c", subcore_axis_name="s")

    @functools.partial(
        pl.kernel,
        mesh=mesh,
        compiler_params=pltpu.CompilerParams(needs_layout_passes=False),
        out_type=jax.ShapeDtypeStruct((nw, n_pad), jnp.float32),
        scratch_types=[
            pltpu.VMEM((2, ch), jnp.int32),
            pltpu.VMEM((ch,), jnp.float32),
            pltpu.VMEM((2, tail), jnp.int32),
            pltpu.VMEM((tail,), jnp.float32),
            pltpu.VMEM((n_pad,), jnp.float32),
            pltpu.SemaphoreType.DMA,
            pltpu.SemaphoreType.DMA,
            pltpu.SemaphoreType.DMA,
            pltpu.SemaphoreType.DMA,
        ],
    )
    def seg_sum(edges_hbm, dist_hbm, out_hbm,
                idx_v, val_v, idx_x, val_x, acc_v, sem1, sem2, sem3, sem4):
        wid = lax.axis_index("s") * _NC + lax.axis_index("c")
        base = wid * ch
        half = ch // 2
        cp_i0 = pltpu.async_copy(
            edges_hbm.at[:, pl.ds(base, half)], idx_v.at[:, pl.ds(0, half)],
            sem1)
        cp_v0 = pltpu.async_copy(
            dist_hbm.at[0, pl.ds(base, half)], val_v.at[pl.ds(0, half)], sem2)
        cp_i1 = pltpu.async_copy(
            edges_hbm.at[:, pl.ds(base + half, half)],
            idx_v.at[:, pl.ds(half, half)], sem3)
        cp_v1 = pltpu.async_copy(
            dist_hbm.at[0, pl.ds(base + half, half)],
            val_v.at[pl.ds(half, half)], sem4)

        @plsc.parallel_loop(0, n_pad, step=_L, unroll=8)
        def zero(i):
            acc_v[pl.ds(i, _L)] = jnp.zeros((_L,), jnp.float32)

        cp_i0.wait()
        cp_v0.wait()

        @plsc.parallel_loop(0, half, step=_L, unroll=8)
        def body0(i):
            plsc.addupdate_scatter(
                acc_v, [idx_v[0, pl.ds(i, _L)]], val_v[pl.ds(i, _L)]
            )

        cp_i1.wait()
        cp_v1.wait()

        @plsc.parallel_loop(half, ch, step=_L, unroll=8)
        def body1(i):
            plsc.addupdate_scatter(
                acc_v, [idx_v[0, pl.ds(i, _L)]], val_v[pl.ds(i, _L)]
            )

        @pl.when(wid == 0)
        def _():
            cpi = pltpu.async_copy(
                edges_hbm.at[:, pl.ds(nw * ch, tail)], idx_x, sem1)
            cpv = pltpu.async_copy(
                dist_hbm.at[0, pl.ds(nw * ch, tail)], val_x, sem2)
            cpi.wait()
            cpv.wait()

            @plsc.parallel_loop(0, tail, step=_L, unroll=8)
            def tail_body(i):
                plsc.addupdate_scatter(
                    acc_v, [idx_x[0, pl.ds(i, _L)]], val_x[pl.ds(i, _L)]
                )

        pltpu.sync_copy(acc_v, out_hbm.at[wid])

    return seg_sum(edges, dist_row)


def _node_update_tc(h, parts, Wn1, b1, Wn2, b2):
    n, d = h.shape
    nw = parts.shape[0]
    B = 2048
    grid = (pl.cdiv(n, B),)

    def body(h_ref, p_ref, Wn1_ref, b1_ref, Wn2_ref, b2_ref, out_ref):
        hb = h_ref[...]
        w1b = Wn1_ref[d:d + 1, :] * 0.01
        C = jnp.broadcast_to(w1b, (nw, d))
        t = jnp.dot(hb, Wn1_ref[:d, :], preferred_element_type=jnp.float32)
        t = t + lax.dot_general(
            p_ref[...], C, (((0,), (0,)), ((), ())),
            preferred_element_type=jnp.float32,
        )
        t = t + b1_ref[...]
        t = t * jax.nn.sigmoid(t)
        o = jnp.dot(t, Wn2_ref[...], preferred_element_type=jnp.float32)
        out_ref[...] = o + b2_ref[...] + hb

    return pl.pallas_call(
        body,
        grid=grid,
        in_specs=[
            pl.BlockSpec((B, d), lambda i: (i, 0)),
            pl.BlockSpec((nw, B), lambda i: (0, i)),
            pl.BlockSpec((d + 1, d), lambda i: (0, 0)),
            pl.BlockSpec((1, d), lambda i: (0, 0)),
            pl.BlockSpec((d, d), lambda i: (0, 0)),
            pl.BlockSpec((1, d), lambda i: (0, 0)),
        ],
        out_specs=pl.BlockSpec((B, d), lambda i: (i, 0)),
        out_shape=jax.ShapeDtypeStruct((n, d), jnp.float32),
    )(h, parts, Wn1, b1.reshape(1, d), Wn2, b2.reshape(1, d))


def kernel(h, edges, distances, W_edg1, b_edg1, W_edg2, b_edg2,
           W_edgi, b_edgi, W_node1, b_node1, W_node2, b_node2):
    n_nodes, d = h.shape
    n_pad = ((n_nodes + 2047) // 2048) * 2048
    parts = _segment_sum_sc(edges.astype(jnp.int32),
                            distances.reshape(1, -1), n_pad)
    return _node_update_tc(h, parts, W_node1, b_node1, W_node2, b_node2)


# speedup vs baseline: 1.0810x; 1.0059x over previous
import functools

import jax
import jax.numpy as jnp
from jax import lax
from jax.experimental import pallas as pl
from jax.experimental.pallas import tpu as pltpu
from jax.experimental.pallas import tpu_sc as plsc

_L = 16
_NC = 2
_NS = 16


def _segment_sum_sc(edges, dist_row, n_pad):
    nw = _NC * _NS
    n_e = dist_row.shape[1]
    ch = n_e // nw // 128 * 128
    tail = n_e - nw * ch
    mesh = plsc.VectorSubcoreMesh(core_axis_name="c", subcore_axis_name="s")

    @functools.partial(
        pl.kernel,
        mesh=mesh,
        compiler_params=pltpu.CompilerParams(needs_layout_passes=False),
        out_type=jax.ShapeDtypeStruct((nw, n_pad), jnp.float32),
        scratch_types=[
            pltpu.VMEM((2, ch), jnp.int32),
            pltpu.VMEM((ch,), jnp.float32),
            pltpu.VMEM((2, tail), jnp.int32),
            pltpu.VMEM((tail,), jnp.float32),
            pltpu.VMEM((n_pad,), jnp.float32),
            pltpu.SemaphoreType.DMA,
            pltpu.SemaphoreType.DMA,
        ],
    )
    def seg_sum(edges_hbm, dist_hbm, out_hbm,
                idx_v, val_v, idx_x, val_x, acc_v, sem1, sem2):
        wid = lax.axis_index("s") * _NC + lax.axis_index("c")
        base = wid * ch
        cp_idx = pltpu.async_copy(
            edges_hbm.at[:, pl.ds(base, ch)], idx_v, sem1)
        cp_val = pltpu.async_copy(
            dist_hbm.at[0, pl.ds(base, ch)], val_v, sem2)

        @plsc.parallel_loop(0, n_pad, step=_L, unroll=8)
        def zero(i):
            acc_v[pl.ds(i, _L)] = jnp.zeros((_L,), jnp.float32)

        cp_idx.wait()
        cp_val.wait()

        @plsc.parallel_loop(0, ch, step=_L, unroll=16)
        def body(i):
            plsc.addupdate_scatter(
                acc_v, [idx_v[0, pl.ds(i, _L)]], val_v[pl.ds(i, _L)]
            )

        @pl.when(wid == 0)
        def _():
            cpi = pltpu.async_copy(
                edges_hbm.at[:, pl.ds(nw * ch, tail)], idx_x, sem1)
            cpv = pltpu.async_copy(
                dist_hbm.at[0, pl.ds(nw * ch, tail)], val_x, sem2)
            cpi.wait()
            cpv.wait()

            @plsc.parallel_loop(0, tail, step=_L, unroll=8)
            def tail_body(i):
                plsc.addupdate_scatter(
                    acc_v, [idx_x[0, pl.ds(i, _L)]], val_x[pl.ds(i, _L)]
                )

        pltpu.sync_copy(acc_v, out_hbm.at[wid])

    return seg_sum(edges, dist_row)


def _node_update_tc(h, parts, Wn1, b1, Wn2, b2):
    n, d = h.shape
    nw = parts.shape[0]
    B = 2048
    grid = (pl.cdiv(n, B),)

    def body(h_ref, p_ref, Wn1_ref, b1_ref, Wn2_ref, b2_ref, out_ref):
        hb = h_ref[...]
        w1b = Wn1_ref[d:d + 1, :] * 0.01
        C = jnp.broadcast_to(w1b, (nw, d))
        t = jnp.dot(hb, Wn1_ref[:d, :], preferred_element_type=jnp.float32)
        t = t + lax.dot_general(
            p_ref[...], C, (((0,), (0,)), ((), ())),
            preferred_element_type=jnp.float32,
        )
        t = t + b1_ref[...]
        t = t * jax.nn.sigmoid(t)
        o = jnp.dot(t, Wn2_ref[...], preferred_element_type=jnp.float32)
        out_ref[...] = o + b2_ref[...] + hb

    return pl.pallas_call(
        body,
        grid=grid,
        in_specs=[
            pl.BlockSpec((B, d), lambda i: (i, 0)),
            pl.BlockSpec((nw, B), lambda i: (0, i)),
            pl.BlockSpec((d + 1, d), lambda i: (0, 0)),
            pl.BlockSpec((1, d), lambda i: (0, 0)),
            pl.BlockSpec((d, d), lambda i: (0, 0)),
            pl.BlockSpec((1, d), lambda i: (0, 0)),
        ],
        out_specs=pl.BlockSpec((B, d), lambda i: (i, 0)),
        out_shape=jax.ShapeDtypeStruct((n, d), jnp.float32),
    )(h, parts, Wn1, b1.reshape(1, d), Wn2, b2.reshape(1, d))


def kernel(h, edges, distances, W_edg1, b_edg1, W_edg2, b_edg2,
           W_edgi, b_edgi, W_node1, b_node1, W_node2, b_node2):
    n_nodes, d = h.shape
    n_pad = ((n_nodes + 2047) // 2048) * 2048
    parts = _segment_sum_sc(edges.astype(jnp.int32),
                            distances.reshape(1, -1), n_pad)
    return _node_update_tc(h, parts, W_node1, b_node1, W_node2, b_node2)


# speedup vs baseline: 1.0877x; 1.0062x over previous
import functools

import jax
import jax.numpy as jnp
from jax import lax
from jax.experimental import pallas as pl
from jax.experimental.pallas import tpu as pltpu
from jax.experimental.pallas import tpu_sc as plsc

_L = 16
_NC = 2
_NS = 16


def _segment_sum_sc(edges, dist_row, n_pad):
    nw = _NC * _NS
    n_e = dist_row.shape[1]
    ch = n_e // nw // 128 * 128
    tail = n_e - nw * ch
    mesh = plsc.VectorSubcoreMesh(core_axis_name="c", subcore_axis_name="s")

    @functools.partial(
        pl.kernel,
        mesh=mesh,
        compiler_params=pltpu.CompilerParams(needs_layout_passes=False),
        out_type=jax.ShapeDtypeStruct((nw, n_pad), jnp.float32),
        scratch_types=[
            pltpu.VMEM((2, ch), jnp.int32),
            pltpu.VMEM((ch,), jnp.float32),
            pltpu.VMEM((2, tail), jnp.int32),
            pltpu.VMEM((tail,), jnp.float32),
            pltpu.VMEM((n_pad,), jnp.float32),
            pltpu.SemaphoreType.DMA,
            pltpu.SemaphoreType.DMA,
        ],
    )
    def seg_sum(edges_hbm, dist_hbm, out_hbm,
                idx_v, val_v, idx_x, val_x, acc_v, sem1, sem2):
        wid = lax.axis_index("s") * _NC + lax.axis_index("c")
        base = wid * ch
        cp_idx = pltpu.async_copy(
            edges_hbm.at[:, pl.ds(base, ch)], idx_v, sem1)
        cp_val = pltpu.async_copy(
            dist_hbm.at[0, pl.ds(base, ch)], val_v, sem2)

        @plsc.parallel_loop(0, n_pad, step=_L, unroll=4)
        def zero(i):
            acc_v[pl.ds(i, _L)] = jnp.zeros((_L,), jnp.float32)

        cp_idx.wait()
        cp_val.wait()

        @plsc.parallel_loop(0, ch, step=_L, unroll=4)
        def body(i):
            plsc.addupdate_scatter(
                acc_v, [idx_v[0, pl.ds(i, _L)]], val_v[pl.ds(i, _L)]
            )

        @pl.when(wid == 0)
        def _():
            cpi = pltpu.async_copy(
                edges_hbm.at[:, pl.ds(nw * ch, tail)], idx_x, sem1)
            cpv = pltpu.async_copy(
                dist_hbm.at[0, pl.ds(nw * ch, tail)], val_x, sem2)
            cpi.wait()
            cpv.wait()

            @plsc.parallel_loop(0, tail, step=_L, unroll=2)
            def tail_body(i):
                plsc.addupdate_scatter(
                    acc_v, [idx_x[0, pl.ds(i, _L)]], val_x[pl.ds(i, _L)]
                )

        pltpu.sync_copy(acc_v, out_hbm.at[wid])

    return seg_sum(edges, dist_row)


def _node_update_tc(h, parts, Wn1, b1, Wn2, b2):
    n, d = h.shape
    nw = parts.shape[0]
    B = 2048
    grid = (pl.cdiv(n, B),)

    def body(h_ref, p_ref, Wn1_ref, b1_ref, Wn2_ref, b2_ref, out_ref):
        hb = h_ref[...]
        w1b = Wn1_ref[d:d + 1, :] * 0.01
        C = jnp.broadcast_to(w1b, (nw, d))
        t = jnp.dot(hb, Wn1_ref[:d, :], preferred_element_type=jnp.float32)
        t = t + lax.dot_general(
            p_ref[...], C, (((0,), (0,)), ((), ())),
            preferred_element_type=jnp.float32,
        )
        t = t + b1_ref[...]
        t = t * jax.nn.sigmoid(t)
        o = jnp.dot(t, Wn2_ref[...], preferred_element_type=jnp.float32)
        out_ref[...] = o + b2_ref[...] + hb

    return pl.pallas_call(
        body,
        grid=grid,
        in_specs=[
            pl.BlockSpec((B, d), lambda i: (i, 0)),
            pl.BlockSpec((nw, B), lambda i: (0, i)),
            pl.BlockSpec((d + 1, d), lambda i: (0, 0)),
            pl.BlockSpec((1, d), lambda i: (0, 0)),
            pl.BlockSpec((d, d), lambda i: (0, 0)),
            pl.BlockSpec((1, d), lambda i: (0, 0)),
        ],
        out_specs=pl.BlockSpec((B, d), lambda i: (i, 0)),
        out_shape=jax.ShapeDtypeStruct((n, d), jnp.float32),
    )(h, parts, Wn1, b1.reshape(1, d), Wn2, b2.reshape(1, d))


def kernel(h, edges, distances, W_edg1, b_edg1, W_edg2, b_edg2,
           W_edgi, b_edgi, W_node1, b_node1, W_node2, b_node2):
    n_nodes, d = h.shape
    n_pad = ((n_nodes + 2047) // 2048) * 2048
    parts = _segment_sum_sc(edges.astype(jnp.int32),
                            distances.reshape(1, -1), n_pad)
    return _node_update_tc(h, parts, W_node1, b_node1, W_node2, b_node2)


# speedup vs baseline: 1.1156x; 1.0256x over previous
import functools

import jax
import jax.numpy as jnp
from jax import lax
from jax.experimental import pallas as pl
from jax.experimental.pallas import tpu as pltpu
from jax.experimental.pallas import tpu_sc as plsc

_L = 16
_NC = 2
_NS = 16


def _segment_sum_sc(edges, dist_row, n_pad):
    nw = _NC * _NS
    n_e = dist_row.shape[1]
    ch = n_e // nw // 128 * 128
    tail = n_e - nw * ch
    mesh = plsc.VectorSubcoreMesh(core_axis_name="c", subcore_axis_name="s")

    @functools.partial(
        pl.kernel,
        mesh=mesh,
        compiler_params=pltpu.CompilerParams(needs_layout_passes=False),
        out_type=jax.ShapeDtypeStruct((nw, n_pad), jnp.float32),
        scratch_types=[
            pltpu.VMEM((2, ch), jnp.int32),
            pltpu.VMEM((ch,), jnp.float32),
            pltpu.VMEM((2, tail), jnp.int32),
            pltpu.VMEM((tail,), jnp.float32),
            pltpu.VMEM((n_pad,), jnp.float32),
            pltpu.SemaphoreType.DMA,
            pltpu.SemaphoreType.DMA,
        ],
    )
    def seg_sum(edges_hbm, dist_hbm, out_hbm,
                idx_v, val_v, idx_x, val_x, acc_v, sem1, sem2):
        wid = lax.axis_index("s") * _NC + lax.axis_index("c")
        base = wid * ch
        cp_idx = pltpu.async_copy(
            edges_hbm.at[:, pl.ds(base, ch)], idx_v, sem1)
        cp_val = pltpu.async_copy(
            dist_hbm.at[0, pl.ds(base, ch)], val_v, sem2)

        @plsc.parallel_loop(0, n_pad, step=_L, unroll=4)
        def zero(i):
            acc_v[pl.ds(i, _L)] = jnp.zeros((_L,), jnp.float32)

        cp_idx.wait()
        cp_val.wait()

        @plsc.parallel_loop(0, ch, step=_L, unroll=4)
        def body(i):
            plsc.addupdate_scatter(
                acc_v, [idx_v[0, pl.ds(i, _L)]], val_v[pl.ds(i, _L)]
            )

        @pl.when(wid == 0)
        def _():
            cpi = pltpu.async_copy(
                edges_hbm.at[:, pl.ds(nw * ch, tail)], idx_x, sem1)
            cpv = pltpu.async_copy(
                dist_hbm.at[0, pl.ds(nw * ch, tail)], val_x, sem2)
            cpi.wait()
            cpv.wait()

            @plsc.parallel_loop(0, tail, step=_L, unroll=2)
            def tail_body(i):
                plsc.addupdate_scatter(
                    acc_v, [idx_x[0, pl.ds(i, _L)]], val_x[pl.ds(i, _L)]
                )

        pltpu.sync_copy(acc_v, out_hbm.at[wid])

    return seg_sum(edges, dist_row)


def _node_update_tc(h, parts, Wn1, b1, Wn2, b2):
    n, d = h.shape
    nw = parts.shape[0]
    B = 2560
    grid = (pl.cdiv(n, B),)

    def body(h_ref, p_ref, Wn1_ref, b1_ref, Wn2_ref, b2_ref, out_ref):
        hb = h_ref[...]
        w1b = Wn1_ref[d:d + 1, :] * 0.01
        C = jnp.broadcast_to(w1b, (nw, d))
        t = jnp.dot(hb, Wn1_ref[:d, :], preferred_element_type=jnp.float32)
        t = t + lax.dot_general(
            p_ref[...], C, (((0,), (0,)), ((), ())),
            preferred_element_type=jnp.float32,
        )
        t = t + b1_ref[...]
        t = t * jax.nn.sigmoid(t)
        o = jnp.dot(t, Wn2_ref[...], preferred_element_type=jnp.float32)
        out_ref[...] = o + b2_ref[...] + hb

    return pl.pallas_call(
        body,
        grid=grid,
        in_specs=[
            pl.BlockSpec((B, d), lambda i: (i, 0)),
            pl.BlockSpec((nw, B), lambda i: (0, i)),
            pl.BlockSpec((d + 1, d), lambda i: (0, 0)),
            pl.BlockSpec((1, d), lambda i: (0, 0)),
            pl.BlockSpec((d, d), lambda i: (0, 0)),
            pl.BlockSpec((1, d), lambda i: (0, 0)),
        ],
        out_specs=pl.BlockSpec((B, d), lambda i: (i, 0)),
        out_shape=jax.ShapeDtypeStruct((n, d), jnp.float32),
    )(h, parts, Wn1, b1.reshape(1, d), Wn2, b2.reshape(1, d))


def kernel(h, edges, distances, W_edg1, b_edg1, W_edg2, b_edg2,
           W_edgi, b_edgi, W_node1, b_node1, W_node2, b_node2):
    n_nodes, d = h.shape
    n_pad = ((n_nodes + 2047) // 2048) * 2048
    parts = _segment_sum_sc(edges.astype(jnp.int32),
                            distances.reshape(1, -1), n_pad)
    return _node_update_tc(h, parts, W_node1, b_node1, W_node2, b_node2)


# speedup vs baseline: 1.1362x; 1.0184x over previous
import functools

import jax
import jax.numpy as jnp
from jax import lax
from jax.experimental import pallas as pl
from jax.experimental.pallas import tpu as pltpu
from jax.experimental.pallas import tpu_sc as plsc

_L = 16
_NC = 2
_NS = 16


def _segment_sum_sc(edges, dist_row, n_pad):
    nw = _NC * _NS
    n_e = dist_row.shape[1]
    ch = n_e // nw // 128 * 128
    tail = n_e - nw * ch
    mesh = plsc.VectorSubcoreMesh(core_axis_name="c", subcore_axis_name="s")

    @functools.partial(
        pl.kernel,
        mesh=mesh,
        compiler_params=pltpu.CompilerParams(needs_layout_passes=False),
        out_type=jax.ShapeDtypeStruct((nw, n_pad), jnp.float32),
        scratch_types=[
            pltpu.VMEM((2, ch), jnp.int32),
            pltpu.VMEM((ch,), jnp.float32),
            pltpu.VMEM((2, tail), jnp.int32),
            pltpu.VMEM((tail,), jnp.float32),
            pltpu.VMEM((n_pad,), jnp.float32),
            pltpu.SemaphoreType.DMA,
            pltpu.SemaphoreType.DMA,
        ],
    )
    def seg_sum(edges_hbm, dist_hbm, out_hbm,
                idx_v, val_v, idx_x, val_x, acc_v, sem1, sem2):
        wid = lax.axis_index("s") * _NC + lax.axis_index("c")
        base = wid * ch
        cp_idx = pltpu.async_copy(
            edges_hbm.at[:, pl.ds(base, ch)], idx_v, sem1)
        cp_val = pltpu.async_copy(
            dist_hbm.at[0, pl.ds(base, ch)], val_v, sem2)

        @plsc.parallel_loop(0, n_pad, step=_L, unroll=4)
        def zero(i):
            acc_v[pl.ds(i, _L)] = jnp.zeros((_L,), jnp.float32)

        cp_idx.wait()
        cp_val.wait()

        @plsc.parallel_loop(0, ch, step=_L, unroll=4)
        def body(i):
            plsc.addupdate_scatter(
                acc_v, [idx_v[0, pl.ds(i, _L)]], val_v[pl.ds(i, _L)]
            )

        @pl.when(wid == 0)
        def _():
            cpi = pltpu.async_copy(
                edges_hbm.at[:, pl.ds(nw * ch, tail)], idx_x, sem1)
            cpv = pltpu.async_copy(
                dist_hbm.at[0, pl.ds(nw * ch, tail)], val_x, sem2)
            cpi.wait()
            cpv.wait()

            @plsc.parallel_loop(0, tail, step=_L, unroll=2)
            def tail_body(i):
                plsc.addupdate_scatter(
                    acc_v, [idx_x[0, pl.ds(i, _L)]], val_x[pl.ds(i, _L)]
                )

        pltpu.sync_copy(acc_v, out_hbm.at[wid])

    return seg_sum(edges, dist_row)


def _node_update_tc(h, parts, Wn1, b1, Wn2, b2):
    n, d = h.shape
    nw = parts.shape[0]
    B = 5120
    grid = (pl.cdiv(n, B),)

    def body(h_ref, p_ref, Wn1_ref, b1_ref, Wn2_ref, b2_ref, out_ref):
        hb = h_ref[...]
        w1b = Wn1_ref[d:d + 1, :] * 0.01
        C = jnp.broadcast_to(w1b, (nw, d))
        t = jnp.dot(hb, Wn1_ref[:d, :], preferred_element_type=jnp.float32)
        t = t + lax.dot_general(
            p_ref[...], C, (((0,), (0,)), ((), ())),
            preferred_element_type=jnp.float32,
        )
        t = t + b1_ref[...]
        t = t * jax.nn.sigmoid(t)
        o = jnp.dot(t, Wn2_ref[...], preferred_element_type=jnp.float32)
        out_ref[...] = o + b2_ref[...] + hb

    return pl.pallas_call(
        body,
        grid=grid,
        in_specs=[
            pl.BlockSpec((B, d), lambda i: (i, 0)),
            pl.BlockSpec((nw, B), lambda i: (0, i)),
            pl.BlockSpec((d + 1, d), lambda i: (0, 0)),
            pl.BlockSpec((1, d), lambda i: (0, 0)),
            pl.BlockSpec((d, d), lambda i: (0, 0)),
            pl.BlockSpec((1, d), lambda i: (0, 0)),
        ],
        out_specs=pl.BlockSpec((B, d), lambda i: (i, 0)),
        out_shape=jax.ShapeDtypeStruct((n, d), jnp.float32),
    )(h, parts, Wn1, b1.reshape(1, d), Wn2, b2.reshape(1, d))


def kernel(h, edges, distances, W_edg1, b_edg1, W_edg2, b_edg2,
           W_edgi, b_edgi, W_node1, b_node1, W_node2, b_node2):
    n_nodes, d = h.shape
    n_pad = ((n_nodes + 2047) // 2048) * 2048
    parts = _segment_sum_sc(edges.astype(jnp.int32),
                            distances.reshape(1, -1), n_pad)
    return _node_update_tc(h, parts, W_node1, b_node1, W_node2, b_node2)
